# baseline (speedup 1.0000x reference)
import functools

import jax
import jax.numpy as jnp
from jax import lax
from jax.experimental import pallas as pl
from jax.experimental.pallas import tpu as pltpu
from jax.experimental.pallas import tpu_sc as plsc

RADIUS2 = 0.25 * 0.25
NSAMPLE = 32
WPQ = 256
QT = 128

NC, NS = 2, 16
NW = NC * NS


def _mask_pack_block(xyz_ref, qT_ref, pk_ref, *, N):
    p2 = xyz_ref[0]
    qT = qT_ref[0]
    d2 = None
    for d in range(3):
        dd = p2[:, d : d + 1] - qT[d : d + 1, :]
        sq = dd * dd
        d2 = sq if d2 is None else d2 + sq
    wfT = (d2 < RADIUS2).astype(jnp.bfloat16)
    g_io = lax.broadcasted_iota(jnp.int32, (N // 16, N), 0)
    i_io = lax.broadcasted_iota(jnp.int32, (N // 16, N), 1)
    pw = lax.shift_left(jnp.int32(1), i_io & 15)
    wt = jnp.where((i_io >> 4) == g_io, pw, 0).astype(jnp.bfloat16)
    pk = lax.dot_general(wt, wfT, (((1,), (0,)), ((), ())),
                         preferred_element_type=jnp.float32)
    pk_ref[0, 0] = pk.astype(jnp.int32)


def _mask_pack(xyz, nqT):
    B, N, _ = xyz.shape
    npoint = nqT.shape[2]
    return pl.pallas_call(
        functools.partial(_mask_pack_block, N=N),
        grid=(B, npoint // QT),
        in_specs=[
            pl.BlockSpec((1, N, 3), lambda b, qt: (b, 0, 0)),
            pl.BlockSpec((1, 3, QT), lambda b, qt: (b, 0, qt)),
        ],
        out_specs=pl.BlockSpec((1, 1, N // 16, QT),
                               lambda b, qt: (b, qt, 0, 0)),
        out_shape=jax.ShapeDtypeStruct((B, npoint // QT, N // 16, QT),
                                       jnp.int32),
    )(xyz, nqT)


def _select_kernel_body(pk_hbm, idx_hbm, pk_v, out_v, *, nq_total):
    wid = lax.axis_index("s") * NC + lax.axis_index("c")
    qpw = nq_total // NW
    nch = qpw // 128
    lanes = lax.broadcasted_iota(jnp.int32, (16,), 0)

    def chunk_body(ch, _):
        ci = wid * nch + ch
        q0 = ci * 128
        pltpu.sync_copy(pk_hbm.at[ci], pk_v)

        def group_body(g, _):
            qlane = g * 16 + lanes
            zero = jnp.zeros((16,), jnp.int32)

            def step(_, carry):
                wptr, wcur, cur, cnt, first = carry
                active = cnt < 32
                adv = (cur == 0) & (wptr < WPQ) & active
                wclamp = jnp.minimum(wptr, WPQ - 1)
                neww = plsc.load_gather(pk_v, [wclamp, qlane])
                wcur = jnp.where(adv, wptr, wcur)
                cur = jnp.where(adv, neww, cur)
                wptr = wptr + adv.astype(jnp.int32)
                emit = active & (cur != 0)
                low = cur & (0 - cur)
                fbits = lax.bitcast_convert_type(low.astype(jnp.float32),
                                                jnp.int32)
                pos = lax.shift_right_logical(fbits, 23) - 127
                pidx = (wcur << 4) + pos
                first = jnp.where(emit & (cnt == 0), pidx, first)
                slot = (jnp.minimum(cnt, 31) << 7) + qlane
                plsc.store_scatter(out_v, [slot], pidx, mask=emit)
                cur = jnp.where(emit, cur & (cur - 1), cur)
                cnt = cnt + emit.astype(jnp.int32)
                return wptr, wcur, cur, cnt, first

            init = (zero, zero, zero, zero, zero)
            _, _, _, cnt, first = lax.fori_loop(0, WPQ + NSAMPLE, step, init)

            def pad(s, _):
                slot = (s << 7) + qlane
                plsc.store_scatter(out_v, [slot], first, mask=s >= cnt)
                return 0

            lax.fori_loop(0, NSAMPLE, pad, 0)
            return 0

        lax.fori_loop(0, 8, group_body, 0)
        pltpu.sync_copy(out_v, idx_hbm.at[pl.ds(q0 * 32, 4096)])
        return 0

    lax.fori_loop(0, nch, chunk_body, 0)


def _select(pk):
    nq_total = pk.shape[0] * 128
    mesh = plsc.VectorSubcoreMesh(core_axis_name="c", subcore_axis_name="s")
    return pl.kernel(
        functools.partial(_select_kernel_body, nq_total=nq_total),
        out_type=jax.ShapeDtypeStruct((nq_total * NSAMPLE,), jnp.int32),
        mesh=mesh,
        compiler_params=pltpu.CompilerParams(needs_layout_passes=False),
        scratch_types=[
            pltpu.VMEM((WPQ, 128), jnp.int32),
            pltpu.VMEM((128 * NSAMPLE,), jnp.int32),
        ],
    )(pk)


def _gather_kernel_body(feat_hbm, idx_hbm, xyzT_hbm, nqT_hbm, out_hbm,
                        table_v, xt_v, ct_v, idx0_v, idx1_v,
                        stage0_v, stage1_v, sem0, sem1, isem,
                        *, B, C, N, npoint):
    wid = lax.axis_index("s") * NC + lax.axis_index("c")
    ncb = C // 8
    tpb = ncb + 1
    ntasks = B * tpb
    QC = 128
    qcn = npoint // QC
    niter = QC * NSAMPLE // 16
    CHW = QC * NSAMPLE

    def idx_src(b, qc):
        return idx_hbm.at[pl.ds((b * npoint + qc * QC) * NSAMPLE, CHW)]

    def drain(stage_v, sem, nch):
        for c in range(nch):
            pltpu.make_async_copy(
                stage_v.at[pl.ds(c * NSAMPLE, NSAMPLE), :],
                out_hbm.at[0, c, :, pl.ds(0, QC)], sem).wait()

    def run_task(b, setup, gbody, nch, chbase):
        setup()
        pltpu.async_copy(idx_src(b, 0), idx0_v, isem)

        def fill_and_send(idx_v, stage_v, sem, qc):
            def gstep(ii, _):
                gbody(idx_v, stage_v, qc, ii * 2)
                gbody(idx_v, stage_v, qc, ii * 2 + 1)
                return 0

            lax.fori_loop(0, niter // 2, gstep, 0)
            for c in range(nch):
                pltpu.async_copy(
                    stage_v.at[pl.ds(c * NSAMPLE, NSAMPLE), :],
                    out_hbm.at[b, chbase + c, :,
                               pl.ds(qc * QC, QC)], sem)

        def qchunk(qc, _):
            def phase(idx_v, stage_v, sem):
                pltpu.make_async_copy(idx_src(b, qc), idx_v, isem).wait()

                @pl.when(qc + 1 < qcn)
                def _():
                    pltpu.async_copy(idx_src(b, qc + 1),
                                     idx1_v if idx_v is idx0_v else idx0_v,
                                     isem)

                @pl.when(qc >= 2)
                def _():
                    drain(stage_v, sem, nch)

                fill_and_send(idx_v, stage_v, sem, qc)

            @pl.when(qc % 2 == 0)
            def _():
                phase(idx0_v, stage0_v, sem0)

            @pl.when(qc % 2 == 1)
            def _():
                phase(idx1_v, stage1_v, sem1)

            return 0

        lax.fori_loop(0, qcn, qchunk, 0)
        drain(stage0_v, sem0, nch)
        drain(stage1_v, sem1, nch)

    def task_body(t, _):
        @pl.when(t < ntasks)
        def _():
            b = t // tpb
            k = t % tpb

            @pl.when(k < ncb)
            def _feature_task():
                def setup():
                    pltpu.sync_copy(feat_hbm.at[(b * C + k * 8) // 8],
                                    table_v)

                def gbody(idx_v, stage_v, qc, i):
                    iv = idx_v[pl.ds(i * 16, 16)]
                    sr = lax.shift_right_logical(i, 3)
                    col = (i % 8) * 16
                    for c in range(8):
                        g = plsc.load_gather(
                            table_v, [jnp.full((16,), c, jnp.int32), iv])
                        stage_v[c * NSAMPLE + sr, pl.ds(col, 16)] = g

                run_task(b, setup, gbody, 8, k * 8)

            @pl.when(k == ncb)
            def _xyz_task():
                def setup():
                    for d in range(3):
                        pltpu.sync_copy(
                            xyzT_hbm.at[pl.ds((b * 3 + d) * N, N)],
                            xt_v.at[pl.ds(d * N, N)])
                        pltpu.sync_copy(
                            nqT_hbm.at[pl.ds((b * 3 + d) * npoint, npoint)],
                            ct_v.at[pl.ds(d * npoint, npoint)])

                def gbody(idx_v, stage_v, qc, i):
                    iv = idx_v[pl.ds(i * 16, 16)]
                    sr = lax.shift_right_logical(i, 3)
                    col = (i % 8) * 16
                    for d in range(3):
                        g = plsc.load_gather(
                            xt_v, [jnp.full((16,), d * N, jnp.int32) + iv])
                        ctr = ct_v[pl.ds(d * npoint + qc * QC + col, 16)]
                        stage_v[d * NSAMPLE + sr, pl.ds(col, 16)] = g - ctr

                run_task(b, setup, gbody, 3, C)

        return 0

    ntask_rounds = -(-(B * tpb) // NW)
    lax.fori_loop(0, ntask_rounds, lambda r, _: task_body(wid + r * NW, _), 0)


def _gather(features, idx, xyzT, nqT, npoint):
    B, C, N = features.shape
    mesh = plsc.VectorSubcoreMesh(core_axis_name="c", subcore_axis_name="s")
    QC = 128
    return pl.kernel(
        functools.partial(_gather_kernel_body, B=B, C=C, N=N,
                          npoint=npoint),
        out_type=jax.ShapeDtypeStruct((B, C + 3, NSAMPLE, npoint),
                                      jnp.float32),
        mesh=mesh,
        compiler_params=pltpu.CompilerParams(needs_layout_passes=False),
        scratch_types=[
            pltpu.VMEM((8, N), jnp.float32),
            pltpu.VMEM((3 * N,), jnp.float32),
            pltpu.VMEM((3 * npoint,), jnp.float32),
            pltpu.VMEM((QC * NSAMPLE,), jnp.int32),
            pltpu.VMEM((QC * NSAMPLE,), jnp.int32),
            pltpu.VMEM((8 * NSAMPLE, QC), jnp.float32),
            pltpu.VMEM((8 * NSAMPLE, QC), jnp.float32),
            pltpu.SemaphoreType.DMA,
            pltpu.SemaphoreType.DMA,
            pltpu.SemaphoreType.DMA,
        ],
    )(features.reshape(B * C // 8, 8, N), idx, xyzT.reshape(B * 3 * N),
      nqT.reshape(B * 3 * npoint))


def kernel(xyz, new_xyz, features):
    B, N, _ = xyz.shape
    npoint = new_xyz.shape[1]
    C = features.shape[1]

    xyzT = jnp.transpose(xyz, (0, 2, 1))
    nqT = jnp.transpose(new_xyz, (0, 2, 1))
    pk = _mask_pack(xyz, nqT)
    idx = _select(pk.reshape(B * npoint // 128, N // 16, 128))
    out_sq = _gather(features, idx, xyzT, nqT, npoint)
    return jnp.transpose(out_sq, (0, 1, 3, 2))


# speedup vs baseline: 1.0812x; 1.0812x over previous
import functools

import jax
import jax.numpy as jnp
from jax import lax
from jax.experimental import pallas as pl
from jax.experimental.pallas import tpu as pltpu
from jax.experimental.pallas import tpu_sc as plsc

RADIUS2 = 0.25 * 0.25
NSAMPLE = 32
WPQ = 256
QT = 128

NC, NS = 2, 16
NW = NC * NS


def _mask_pack_block(xyzT_ref, q_ref, pk_ref, *, N):
    pT = xyzT_ref[0]
    q = q_ref[0]
    d2 = None
    for d in range(3):
        dd = q[:, d : d + 1] - pT[d : d + 1, :]
        sq = dd * dd
        d2 = sq if d2 is None else d2 + sq
    wf = (d2 < RADIUS2).astype(jnp.bfloat16)
    i_io = lax.broadcasted_iota(jnp.int32, (N, N // 16), 0)
    g_io = lax.broadcasted_iota(jnp.int32, (N, N // 16), 1)
    pw = lax.shift_left(jnp.int32(1), i_io & 15)
    wt = jnp.where((i_io >> 4) == g_io, pw, 0).astype(jnp.bfloat16)
    pk = lax.dot_general(wf, wt, (((1,), (0,)), ((), ())),
                         preferred_element_type=jnp.float32)
    pk_ref[0] = pk.astype(jnp.int32)


def _mask_pack(xyz, new_xyz):
    B, N, _ = xyz.shape
    npoint = new_xyz.shape[1]
    xyzT = jnp.transpose(xyz, (0, 2, 1))
    return pl.pallas_call(
        functools.partial(_mask_pack_block, N=N),
        grid=(B, npoint // QT),
        in_specs=[
            pl.BlockSpec((1, 3, N), lambda b, qt: (b, 0, 0)),
            pl.BlockSpec((1, QT, 3), lambda b, qt: (b, qt, 0)),
        ],
        out_specs=pl.BlockSpec((1, QT, N // 16), lambda b, qt: (b, qt, 0)),
        out_shape=jax.ShapeDtypeStruct((B, npoint, N // 16), jnp.int32),
    )(xyzT, new_xyz)


def _select_kernel_body(pk_hbm, idx_hbm, pk_v, out_v, *, nq_total):
    wid = lax.axis_index("s") * NC + lax.axis_index("c")
    qpw = nq_total // NW
    nch = qpw // 128
    lanes = lax.broadcasted_iota(jnp.int32, (16,), 0)

    def chunk_body(ch, _):
        q0 = wid * qpw + ch * 128
        pltpu.sync_copy(pk_hbm.at[pl.ds(q0 * WPQ, 128 * WPQ)], pk_v)

        def group_body(g, _):
            qlane = g * 16 + lanes
            zero = jnp.zeros((16,), jnp.int32)

            def step(_, carry):
                wptr, wcur, cur, cnt, first = carry
                active = cnt < 32
                adv = (cur == 0) & (wptr < WPQ) & active
                wclamp = jnp.minimum(wptr, WPQ - 1)
                neww = plsc.load_gather(pk_v, [qlane * WPQ + wclamp])
                wcur = jnp.where(adv, wptr, wcur)
                cur = jnp.where(adv, neww, cur)
                wptr = wptr + adv.astype(jnp.int32)
                emit = active & (cur != 0)
                low = cur & (0 - cur)
                fbits = lax.bitcast_convert_type(low.astype(jnp.float32),
                                                jnp.int32)
                pos = lax.shift_right_logical(fbits, 23) - 127
                pidx = (wcur << 4) + pos
                first = jnp.where(emit & (cnt == 0), pidx, first)
                slot = (jnp.minimum(cnt, 31) << 7) + qlane
                plsc.store_scatter(out_v, [slot], pidx, mask=emit)
                cur = jnp.where(emit, cur & (cur - 1), cur)
                cnt = cnt + emit.astype(jnp.int32)
                return wptr, wcur, cur, cnt, first

            init = (zero, zero, zero, zero, zero)
            _, _, _, cnt, first = lax.fori_loop(0, WPQ + NSAMPLE, step, init)

            def pad(s, _):
                slot = (s << 7) + qlane
                plsc.store_scatter(out_v, [slot], first, mask=s >= cnt)
                return 0

            lax.fori_loop(0, NSAMPLE, pad, 0)
            return 0

        lax.fori_loop(0, 8, group_body, 0)
        pltpu.sync_copy(out_v, idx_hbm.at[pl.ds(q0 * 32, 4096)])
        return 0

    lax.fori_loop(0, nch, chunk_body, 0)


def _select(pk):
    nq_total = pk.shape[0]
    pk = pk.reshape(nq_total * WPQ)
    mesh = plsc.VectorSubcoreMesh(core_axis_name="c", subcore_axis_name="s")
    return pl.kernel(
        functools.partial(_select_kernel_body, nq_total=nq_total),
        out_type=jax.ShapeDtypeStruct((nq_total * NSAMPLE,), jnp.int32),
        mesh=mesh,
        compiler_params=pltpu.CompilerParams(needs_layout_passes=False),
        scratch_types=[
            pltpu.VMEM((128 * WPQ,), jnp.int32),
            pltpu.VMEM((128 * NSAMPLE,), jnp.int32),
        ],
    )(pk)


def _gather_kernel_body(feat_hbm, idx_hbm, xyzT_hbm, nqT_hbm, out_hbm,
                        table_v, xt_v, ct_v, idx0_v, idx1_v,
                        stage0_v, stage1_v, sem0, sem1, isem,
                        *, B, C, N, npoint):
    wid = lax.axis_index("s") * NC + lax.axis_index("c")
    ncb = C // 8
    tpb = ncb + 1
    ntasks = B * tpb
    QC = 128
    qcn = npoint // QC
    niter = QC * NSAMPLE // 16
    CHW = QC * NSAMPLE

    def idx_src(b, qc):
        return idx_hbm.at[pl.ds((b * npoint + qc * QC) * NSAMPLE, CHW)]

    def drain(stage_v, sem, nch):
        for c in range(nch):
            pltpu.make_async_copy(
                stage_v.at[pl.ds(c * NSAMPLE, NSAMPLE), :],
                out_hbm.at[0, c, :, pl.ds(0, QC)], sem).wait()

    def run_task(b, setup, gbody, nch, chbase):
        setup()
        pltpu.async_copy(idx_src(b, 0), idx0_v, isem)

        def fill_and_send(idx_v, stage_v, sem, qc):
            def gstep(ii, _):
                gbody(idx_v, stage_v, qc, ii * 2)
                gbody(idx_v, stage_v, qc, ii * 2 + 1)
                return 0

            lax.fori_loop(0, niter // 2, gstep, 0)
            for c in range(nch):
                pltpu.async_copy(
                    stage_v.at[pl.ds(c * NSAMPLE, NSAMPLE), :],
                    out_hbm.at[b, chbase + c, :,
                               pl.ds(qc * QC, QC)], sem)

        def qchunk(qc, _):
            def phase(idx_v, stage_v, sem):
                pltpu.make_async_copy(idx_src(b, qc), idx_v, isem).wait()

                @pl.when(qc + 1 < qcn)
                def _():
                    pltpu.async_copy(idx_src(b, qc + 1),
                                     idx1_v if idx_v is idx0_v else idx0_v,
                                     isem)

                @pl.when(qc >= 2)
                def _():
                    drain(stage_v, sem, nch)

                fill_and_send(idx_v, stage_v, sem, qc)

            @pl.when(qc % 2 == 0)
            def _():
                phase(idx0_v, stage0_v, sem0)

            @pl.when(qc % 2 == 1)
            def _():
                phase(idx1_v, stage1_v, sem1)

            return 0

        lax.fori_loop(0, qcn, qchunk, 0)
        drain(stage0_v, sem0, nch)
        drain(stage1_v, sem1, nch)

    def task_body(t, _):
        @pl.when(t < ntasks)
        def _():
            b = t // tpb
            k = t % tpb

            @pl.when(k < ncb)
            def _feature_task():
                def setup():
                    pltpu.sync_copy(feat_hbm.at[(b * C + k * 8) // 8],
                                    table_v)

                def gbody(idx_v, stage_v, qc, i):
                    iv = idx_v[pl.ds(i * 16, 16)]
                    sr = lax.shift_right_logical(i, 3)
                    col = (i % 8) * 16
                    for c in range(8):
                        g = plsc.load_gather(
                            table_v, [jnp.full((16,), c, jnp.int32), iv])
                        stage_v[c * NSAMPLE + sr, pl.ds(col, 16)] = g

                run_task(b, setup, gbody, 8, k * 8)

            @pl.when(k == ncb)
            def _xyz_task():
                def setup():
                    for d in range(3):
                        pltpu.sync_copy(
                            xyzT_hbm.at[pl.ds((b * 3 + d) * N, N)],
                            xt_v.at[pl.ds(d * N, N)])
                        pltpu.sync_copy(
                            nqT_hbm.at[pl.ds((b * 3 + d) * npoint, npoint)],
                            ct_v.at[pl.ds(d * npoint, npoint)])

                def gbody(idx_v, stage_v, qc, i):
                    iv = idx_v[pl.ds(i * 16, 16)]
                    sr = lax.shift_right_logical(i, 3)
                    col = (i % 8) * 16
                    for d in range(3):
                        g = plsc.load_gather(
                            xt_v, [jnp.full((16,), d * N, jnp.int32) + iv])
                        ctr = ct_v[pl.ds(d * npoint + qc * QC + col, 16)]
                        stage_v[d * NSAMPLE + sr, pl.ds(col, 16)] = g - ctr

                run_task(b, setup, gbody, 3, C)

        return 0

    ntask_rounds = -(-(B * tpb) // NW)
    lax.fori_loop(0, ntask_rounds, lambda r, _: task_body(wid + r * NW, _), 0)


def _gather(features, idx, xyzT, nqT, npoint):
    B, C, N = features.shape
    mesh = plsc.VectorSubcoreMesh(core_axis_name="c", subcore_axis_name="s")
    QC = 128
    return pl.kernel(
        functools.partial(_gather_kernel_body, B=B, C=C, N=N,
                          npoint=npoint),
        out_type=jax.ShapeDtypeStruct((B, C + 3, NSAMPLE, npoint),
                                      jnp.float32),
        mesh=mesh,
        compiler_params=pltpu.CompilerParams(needs_layout_passes=False),
        scratch_types=[
            pltpu.VMEM((8, N), jnp.float32),
            pltpu.VMEM((3 * N,), jnp.float32),
            pltpu.VMEM((3 * npoint,), jnp.float32),
            pltpu.VMEM((QC * NSAMPLE,), jnp.int32),
            pltpu.VMEM((QC * NSAMPLE,), jnp.int32),
            pltpu.VMEM((8 * NSAMPLE, QC), jnp.float32),
            pltpu.VMEM((8 * NSAMPLE, QC), jnp.float32),
            pltpu.SemaphoreType.DMA,
            pltpu.SemaphoreType.DMA,
            pltpu.SemaphoreType.DMA,
        ],
    )(features.reshape(B * C // 8, 8, N), idx, xyzT.reshape(B * 3 * N),
      nqT.reshape(B * 3 * npoint))


def kernel(xyz, new_xyz, features):
    B, N, _ = xyz.shape
    npoint = new_xyz.shape[1]
    C = features.shape[1]

    pk = _mask_pack(xyz, new_xyz)
    idx = _select(pk.reshape(B * npoint, N // 16))
    xyzT = jnp.transpose(xyz, (0, 2, 1))
    nqT = jnp.transpose(new_xyz, (0, 2, 1))
    out_sq = _gather(features, idx, xyzT, nqT, npoint)
    return jnp.transpose(out_sq, (0, 1, 3, 2))


# speedup vs baseline: 1.0843x; 1.0028x over previous
import functools

import jax
import jax.numpy as jnp
from jax import lax
from jax.experimental import pallas as pl
from jax.experimental.pallas import tpu as pltpu
from jax.experimental.pallas import tpu_sc as plsc

RADIUS2 = 0.25 * 0.25
NSAMPLE = 32
WPQ = 256
QT = 128

NC, NS = 2, 16
NW = NC * NS


def _mask_pack_block(xyzT_ref, q_ref, pk_ref, *, N):
    pT = xyzT_ref[0]
    q = q_ref[0]
    d2 = None
    for d in range(3):
        dd = q[:, d : d + 1] - pT[d : d + 1, :]
        sq = dd * dd
        d2 = sq if d2 is None else d2 + sq
    wf = (d2 < RADIUS2).astype(jnp.bfloat16)
    i_io = lax.broadcasted_iota(jnp.int32, (N, N // 16), 0)
    g_io = lax.broadcasted_iota(jnp.int32, (N, N // 16), 1)
    pw = lax.shift_left(jnp.int32(1), i_io & 15)
    wt = jnp.where((i_io >> 4) == g_io, pw, 0).astype(jnp.bfloat16)
    pk = lax.dot_general(wf, wt, (((1,), (0,)), ((), ())),
                         preferred_element_type=jnp.float32)
    pk_ref[0] = pk.astype(jnp.int32)


def _mask_pack(xyz, new_xyz):
    B, N, _ = xyz.shape
    npoint = new_xyz.shape[1]
    xyzT = jnp.transpose(xyz, (0, 2, 1))
    return pl.pallas_call(
        functools.partial(_mask_pack_block, N=N),
        grid=(B, npoint // QT),
        in_specs=[
            pl.BlockSpec((1, 3, N), lambda b, qt: (b, 0, 0)),
            pl.BlockSpec((1, QT, 3), lambda b, qt: (b, qt, 0)),
        ],
        out_specs=pl.BlockSpec((1, QT, N // 16), lambda b, qt: (b, qt, 0)),
        out_shape=jax.ShapeDtypeStruct((B, npoint, N // 16), jnp.int32),
    )(xyzT, new_xyz)


def _select_kernel_body(pk_hbm, idx_hbm, pk_v, out_v, *, nq_total):
    wid = lax.axis_index("s") * NC + lax.axis_index("c")
    qpw = nq_total // NW
    nch = qpw // 128
    lanes = lax.broadcasted_iota(jnp.int32, (16,), 0)

    def chunk_body(ch, _):
        q0 = wid * qpw + ch * 128
        pltpu.sync_copy(pk_hbm.at[pl.ds(q0 * WPQ, 128 * WPQ)], pk_v)

        def group_body(g, _):
            qlane = g * 16 + lanes
            zero = jnp.zeros((16,), jnp.int32)

            def step(_, carry):
                wptr, wcur, cur, cnt, first = carry
                active = cnt < 32
                adv = (cur == 0) & (wptr < WPQ) & active
                wclamp = jnp.minimum(wptr, WPQ - 1)
                neww = plsc.load_gather(pk_v, [qlane * WPQ + wclamp])
                wcur = jnp.where(adv, wptr, wcur)
                cur = jnp.where(adv, neww, cur)
                wptr = wptr + adv.astype(jnp.int32)
                emit = active & (cur != 0)
                low = cur & (0 - cur)
                fbits = lax.bitcast_convert_type(low.astype(jnp.float32),
                                                jnp.int32)
                pos = lax.shift_right_logical(fbits, 23) - 127
                pidx = (wcur << 4) + pos
                first = jnp.where(emit & (cnt == 0), pidx, first)
                slot = (jnp.minimum(cnt, 31) << 7) + qlane
                plsc.store_scatter(out_v, [slot], pidx, mask=emit)
                cur = jnp.where(emit, cur & (cur - 1), cur)
                cnt = cnt + emit.astype(jnp.int32)
                return wptr, wcur, cur, cnt, first

            init = (zero, zero, zero, zero, zero)
            _, _, _, cnt, first = lax.fori_loop(0, WPQ + NSAMPLE, step, init)

            def pad(s, _):
                slot = (s << 7) + qlane
                plsc.store_scatter(out_v, [slot], first, mask=s >= cnt)
                return 0

            lax.fori_loop(0, NSAMPLE, pad, 0)
            return 0

        lax.fori_loop(0, 8, group_body, 0)
        pltpu.sync_copy(out_v, idx_hbm.at[pl.ds(q0 * 32, 4096)])
        return 0

    lax.fori_loop(0, nch, chunk_body, 0)


def _select(pk):
    nq_total = pk.shape[0]
    pk = pk.reshape(nq_total * WPQ)
    mesh = plsc.VectorSubcoreMesh(core_axis_name="c", subcore_axis_name="s")
    return pl.kernel(
        functools.partial(_select_kernel_body, nq_total=nq_total),
        out_type=jax.ShapeDtypeStruct((nq_total * NSAMPLE,), jnp.int32),
        mesh=mesh,
        compiler_params=pltpu.CompilerParams(needs_layout_passes=False),
        scratch_types=[
            pltpu.VMEM((128 * WPQ,), jnp.int32),
            pltpu.VMEM((128 * NSAMPLE,), jnp.int32),
        ],
    )(pk)


def _gather_kernel_body(feat_hbm, idx_hbm, xyzT_hbm, nqT_hbm, out_hbm,
                        table_v, xt_v, ct_v, idx0_v, idx1_v,
                        stage0_v, stage1_v, sem0, sem1, isem,
                        *, B, C, N, npoint):
    wid = lax.axis_index("s") * NC + lax.axis_index("c")
    ncb = C // 8
    tpb = ncb + 1
    ntasks = B * tpb
    QC = 128
    qcn = npoint // QC
    niter = QC * NSAMPLE // 16
    CHW = QC * NSAMPLE

    def idx_src(b, qc):
        return idx_hbm.at[pl.ds((b * npoint + qc * QC) * NSAMPLE, CHW)]

    def drain(stage_v, sem, nch):
        for c in range(nch):
            pltpu.make_async_copy(
                stage_v.at[pl.ds(c * NSAMPLE, NSAMPLE), :],
                out_hbm.at[0, c, :, pl.ds(0, QC)], sem).wait()

    def run_task(b, setup, gbody, nch, chbase):
        setup()
        pltpu.async_copy(idx_src(b, 0), idx0_v, isem)

        def fill_and_send(idx_v, stage_v, sem, qc):
            def gstep(ii, _):
                for u in range(4):
                    gbody(idx_v, stage_v, qc, ii * 4 + u)
                return 0

            lax.fori_loop(0, niter // 4, gstep, 0)
            for c in range(nch):
                pltpu.async_copy(
                    stage_v.at[pl.ds(c * NSAMPLE, NSAMPLE), :],
                    out_hbm.at[b, chbase + c, :,
                               pl.ds(qc * QC, QC)], sem)

        def qchunk(qc, _):
            def phase(idx_v, stage_v, sem):
                pltpu.make_async_copy(idx_src(b, qc), idx_v, isem).wait()

                @pl.when(qc + 1 < qcn)
                def _():
                    pltpu.async_copy(idx_src(b, qc + 1),
                                     idx1_v if idx_v is idx0_v else idx0_v,
                                     isem)

                @pl.when(qc >= 2)
                def _():
                    drain(stage_v, sem, nch)

                fill_and_send(idx_v, stage_v, sem, qc)

            @pl.when(qc % 2 == 0)
            def _():
                phase(idx0_v, stage0_v, sem0)

            @pl.when(qc % 2 == 1)
            def _():
                phase(idx1_v, stage1_v, sem1)

            return 0

        lax.fori_loop(0, qcn, qchunk, 0)
        drain(stage0_v, sem0, nch)
        drain(stage1_v, sem1, nch)

    def task_body(t, _):
        @pl.when(t < ntasks)
        def _():
            b = t // tpb
            k = t % tpb

            @pl.when(k < ncb)
            def _feature_task():
                def setup():
                    pltpu.sync_copy(feat_hbm.at[(b * C + k * 8) // 8],
                                    table_v)

                def gbody(idx_v, stage_v, qc, i):
                    iv = idx_v[pl.ds(i * 16, 16)]
                    sr = lax.shift_right_logical(i, 3)
                    col = (i % 8) * 16
                    for c in range(8):
                        g = plsc.load_gather(
                            table_v, [jnp.full((16,), c, jnp.int32), iv])
                        stage_v[c * NSAMPLE + sr, pl.ds(col, 16)] = g

                run_task(b, setup, gbody, 8, k * 8)

            @pl.when(k == ncb)
            def _xyz_task():
                def setup():
                    for d in range(3):
                        pltpu.sync_copy(
                            xyzT_hbm.at[pl.ds((b * 3 + d) * N, N)],
                            xt_v.at[pl.ds(d * N, N)])
                        pltpu.sync_copy(
                            nqT_hbm.at[pl.ds((b * 3 + d) * npoint, npoint)],
                            ct_v.at[pl.ds(d * npoint, npoint)])

                def gbody(idx_v, stage_v, qc, i):
                    iv = idx_v[pl.ds(i * 16, 16)]
                    sr = lax.shift_right_logical(i, 3)
                    col = (i % 8) * 16
                    for d in range(3):
                        g = plsc.load_gather(
                            xt_v, [jnp.full((16,), d * N, jnp.int32) + iv])
                        ctr = ct_v[pl.ds(d * npoint + qc * QC + col, 16)]
                        stage_v[d * NSAMPLE + sr, pl.ds(col, 16)] = g - ctr

                run_task(b, setup, gbody, 3, C)

        return 0

    ntask_rounds = -(-(B * tpb) // NW)
    lax.fori_loop(0, ntask_rounds, lambda r, _: task_body(wid + r * NW, _), 0)


def _gather(features, idx, xyzT, nqT, npoint):
    B, C, N = features.shape
    mesh = plsc.VectorSubcoreMesh(core_axis_name="c", subcore_axis_name="s")
    QC = 128
    return pl.kernel(
        functools.partial(_gather_kernel_body, B=B, C=C, N=N,
                          npoint=npoint),
        out_type=jax.ShapeDtypeStruct((B, C + 3, NSAMPLE, npoint),
                                      jnp.float32),
        mesh=mesh,
        compiler_params=pltpu.CompilerParams(needs_layout_passes=False),
        scratch_types=[
            pltpu.VMEM((8, N), jnp.float32),
            pltpu.VMEM((3 * N,), jnp.float32),
            pltpu.VMEM((3 * npoint,), jnp.float32),
            pltpu.VMEM((QC * NSAMPLE,), jnp.int32),
            pltpu.VMEM((QC * NSAMPLE,), jnp.int32),
            pltpu.VMEM((8 * NSAMPLE, QC), jnp.float32),
            pltpu.VMEM((8 * NSAMPLE, QC), jnp.float32),
            pltpu.SemaphoreType.DMA,
            pltpu.SemaphoreType.DMA,
            pltpu.SemaphoreType.DMA,
        ],
    )(features.reshape(B * C // 8, 8, N), idx, xyzT.reshape(B * 3 * N),
      nqT.reshape(B * 3 * npoint))


def kernel(xyz, new_xyz, features):
    B, N, _ = xyz.shape
    npoint = new_xyz.shape[1]
    C = features.shape[1]

    pk = _mask_pack(xyz, new_xyz)
    idx = _select(pk.reshape(B * npoint, N // 16))
    xyzT = jnp.transpose(xyz, (0, 2, 1))
    nqT = jnp.transpose(new_xyz, (0, 2, 1))
    out_sq = _gather(features, idx, xyzT, nqT, npoint)
    return jnp.transpose(out_sq, (0, 1, 3, 2))
